# baseline (speedup 1.0000x reference)
import functools

import jax
import jax.numpy as jnp
from jax import lax
from jax.experimental import pallas as pl
from jax.experimental.pallas import tpu as pltpu
from jax.experimental.pallas import tpu_sc as plsc

ROWS = 16384
COLS = 200
NSEL = 100

_info = plsc.get_sparse_core_info()
NC, NS, L = _info.num_cores, _info.num_subcores, _info.num_lanes
NW = NC * NS
ROWS_PER_W = ROWS // NW
CHUNK = 128
NCHUNK = ROWS_PER_W // CHUNK
GROUP = 4
NVEC = GROUP * NSEL // L
NGROUP = CHUNK // GROUP

_mesh = plsc.VectorSubcoreMesh(core_axis_name="c", subcore_axis_name="s")


@functools.partial(
    pl.kernel,
    mesh=_mesh,
    out_type=jax.ShapeDtypeStruct((NW * NCHUNK, CHUNK * NSEL), jnp.float32),
    scratch_types=[
        pltpu.VMEM((NSEL,), jnp.int32),
        pltpu.VMEM((CHUNK, COLS), jnp.float32),
        pltpu.VMEM((CHUNK * NSEL,), jnp.float32),
    ],
    compiler_params=pltpu.CompilerParams(
        needs_layout_passes=False, use_tc_tiling_on_sc=False
    ),
)
def _sc_gather(in_hbm, idx_hbm, out_hbm, idx_v, rows_v, out_v):
    wid = lax.axis_index("s") * NC + lax.axis_index("c")
    base_row = wid * ROWS_PER_W

    pltpu.sync_copy(idx_hbm, idx_v)
    lanes = lax.iota(jnp.int32, L)
    rpat, cpat = [], []
    for v in range(NVEC):
        p = lanes + v * L
        rpat.append(p // NSEL)
        cpat.append(plsc.load_gather(idx_v, [p % NSEL]))

    for chunk in range(NCHUNK):
        slab = wid * NCHUNK + chunk
        row0 = base_row + chunk * CHUNK
        pltpu.sync_copy(in_hbm.at[pl.ds(row0, CHUNK), :], rows_v)

        def body(g, carry):
            rbase = g * GROUP
            dst = g * (GROUP * NSEL)
            for v in range(NVEC):
                out_v[pl.ds(pl.multiple_of(dst + v * L, L), L)] = (
                    plsc.load_gather(rows_v, [rpat[v] + rbase, cpat[v]])
                )
            return carry

        lax.fori_loop(0, NGROUP, body, 0)
        pltpu.sync_copy(out_v, out_hbm.at[slab])


def kernel(inputs, categorical_idx):
    packed = _sc_gather(inputs, categorical_idx)
    return packed.reshape(ROWS, NSEL)


# speedup vs baseline: 1.2822x; 1.2822x over previous
import functools

import jax
import jax.numpy as jnp
from jax import lax
from jax.experimental import pallas as pl
from jax.experimental.pallas import tpu as pltpu
from jax.experimental.pallas import tpu_sc as plsc

ROWS = 16384
COLS = 200
NSEL = 100

_info = plsc.get_sparse_core_info()
NC, NS, L = _info.num_cores, _info.num_subcores, _info.num_lanes
NW = NC * NS
ROWS_PER_W = ROWS // NW
CHUNK = 128
NCHUNK = ROWS_PER_W // CHUNK
GROUP = 4
NVEC = GROUP * NSEL // L
NGROUP = CHUNK // GROUP

_mesh = plsc.VectorSubcoreMesh(core_axis_name="c", subcore_axis_name="s")


@functools.partial(
    pl.kernel,
    mesh=_mesh,
    out_type=jax.ShapeDtypeStruct((NW * NCHUNK, CHUNK * NSEL), jnp.float32),
    scratch_types=[
        pltpu.VMEM((NSEL,), jnp.int32),
        pltpu.VMEM((CHUNK, COLS), jnp.float32),
        pltpu.VMEM((CHUNK * NSEL,), jnp.float32),
    ],
    compiler_params=pltpu.CompilerParams(
        needs_layout_passes=False, use_tc_tiling_on_sc=True
    ),
)
def _sc_gather(in_hbm, idx_hbm, out_hbm, idx_v, rows_v, out_v):
    wid = lax.axis_index("s") * NC + lax.axis_index("c")
    base_row = wid * ROWS_PER_W

    pltpu.sync_copy(idx_hbm, idx_v)
    lanes = lax.iota(jnp.int32, L)
    rpat, cpat = [], []
    for v in range(NVEC):
        p = lanes + v * L
        rpat.append(p // NSEL)
        cpat.append(plsc.load_gather(idx_v, [p % NSEL]))

    for chunk in range(NCHUNK):
        slab = wid * NCHUNK + chunk
        row0 = base_row + chunk * CHUNK
        pltpu.sync_copy(in_hbm.at[pl.ds(row0, CHUNK), :], rows_v)

        def body(g, carry):
            rbase = g * GROUP
            dst = g * (GROUP * NSEL)
            for v in range(NVEC):
                out_v[pl.ds(pl.multiple_of(dst + v * L, L), L)] = (
                    plsc.load_gather(rows_v, [rpat[v] + rbase, cpat[v]])
                )
            return carry

        lax.fori_loop(0, NGROUP, body, 0)
        pltpu.sync_copy(out_v, out_hbm.at[slab])


def kernel(inputs, categorical_idx):
    packed = _sc_gather(inputs, categorical_idx)
    return packed.reshape(ROWS, NSEL)


# speedup vs baseline: 2.7251x; 2.1253x over previous
import functools

import jax
import jax.numpy as jnp
from jax import lax
from jax.experimental import pallas as pl
from jax.experimental.pallas import tpu as pltpu
from jax.experimental.pallas import tpu_sc as plsc

ROWS = 16384
COLS = 200
NSEL = 100

_info = plsc.get_sparse_core_info()
NC, NS, L = _info.num_cores, _info.num_subcores, _info.num_lanes
NW = NC * NS
NBLK = 7
NPAD = NBLK * L
SPLIT = 32
SEG = ROWS // SPLIT
TASKS_PER_W = NBLK * SPLIT // NW

_mesh = plsc.VectorSubcoreMesh(core_axis_name="c", subcore_axis_name="s")


@functools.partial(
    pl.kernel,
    mesh=_mesh,
    out_type=jax.ShapeDtypeStruct((NPAD, ROWS), jnp.float32),
    scratch_types=[
        pltpu.VMEM((NSEL,), jnp.int32),
        pltpu.VMEM((L, SEG), jnp.float32),
        pltpu.SemaphoreType.DMA,
        pltpu.SemaphoreType.DMA,
    ],
    compiler_params=pltpu.CompilerParams(
        needs_layout_passes=False, use_tc_tiling_on_sc=True
    ),
)
def _sc_rowgather(inT_hbm, idx_hbm, outT_hbm, idx_v, buf, gsem, ssem):
    wid = lax.axis_index("s") * NC + lax.axis_index("c")
    pltpu.sync_copy(idx_hbm, idx_v)
    lanes = lax.iota(jnp.int32, L)

    for k in range(TASKS_PER_W):
        t = wid * TASKS_PER_W + k
        blk = t % NBLK
        seg = t // NBLK
        j0 = blk * L
        col0 = pl.multiple_of(seg * SEG, SEG)
        rows = plsc.load_gather(idx_v, [jnp.minimum(j0 + lanes, NSEL - 1)])
        pltpu.make_async_copy(
            inT_hbm.at[rows, pl.ds(col0, SEG)], buf, gsem
        ).start()
        pltpu.make_async_copy(
            inT_hbm.at[rows, pl.ds(col0, SEG)], buf, gsem
        ).wait()
        pltpu.make_async_copy(
            buf, outT_hbm.at[pl.ds(j0, L), pl.ds(col0, SEG)], ssem
        ).start()
        pltpu.make_async_copy(
            buf, outT_hbm.at[pl.ds(j0, L), pl.ds(col0, SEG)], ssem
        ).wait()


def kernel(inputs, categorical_idx):
    outT = _sc_rowgather(inputs.T, categorical_idx)
    return outT.T[:, :NSEL]


# speedup vs baseline: 3.4287x; 1.2582x over previous
import functools

import jax
import jax.numpy as jnp
from jax import lax
from jax.experimental import pallas as pl
from jax.experimental.pallas import tpu as pltpu
from jax.experimental.pallas import tpu_sc as plsc

ROWS = 16384
COLS = 200
NSEL = 100

_info = plsc.get_sparse_core_info()
NC, NS, L = _info.num_cores, _info.num_subcores, _info.num_lanes
NW = NC * NS
NBLK = NSEL // L
NTAIL = NSEL - NBLK * L
SPLIT = 32
SEG = ROWS // SPLIT
FULL_PER_W = NBLK * SPLIT // NW

_mesh = plsc.VectorSubcoreMesh(core_axis_name="c", subcore_axis_name="s")


@functools.partial(
    pl.kernel,
    mesh=_mesh,
    out_type=jax.ShapeDtypeStruct((NSEL, ROWS), jnp.float32),
    scratch_types=[
        pltpu.VMEM((NSEL,), jnp.int32),
        pltpu.VMEM((L, SEG), jnp.float32),
        pltpu.VMEM((L, SEG), jnp.float32),
        pltpu.SemaphoreType.DMA,
        pltpu.SemaphoreType.DMA,
        pltpu.SemaphoreType.DMA,
    ],
    compiler_params=pltpu.CompilerParams(
        needs_layout_passes=False, use_tc_tiling_on_sc=True
    ),
)
def _sc_rowgather(inT_hbm, idx_hbm, outT_hbm, idx_v, buf0, buf1, gsem, ssem0, ssem1):
    wid = lax.axis_index("s") * NC + lax.axis_index("c")
    pltpu.sync_copy(idx_hbm, idx_v)
    lanes = lax.iota(jnp.int32, L)
    bufs = (buf0, buf1)
    ssems = (ssem0, ssem1)
    pending = [None, None]

    def gather_rows(j0, col0, buf):
        rows = plsc.load_gather(idx_v, [jnp.minimum(j0 + lanes, NSEL - 1)])
        cp = pltpu.make_async_copy(
            inT_hbm.at[rows, pl.ds(col0, SEG)], buf, gsem
        )
        cp.start()
        cp.wait()

    for k in range(FULL_PER_W):
        b = k % 2
        t = wid * FULL_PER_W + k
        blk = t % NBLK
        seg = t // NBLK
        j0 = blk * L
        col0 = pl.multiple_of(seg * SEG, SEG)
        if pending[b] is not None:
            pending[b].wait()
        gather_rows(j0, col0, bufs[b])
        sc = pltpu.make_async_copy(
            bufs[b], outT_hbm.at[pl.ds(j0, L), pl.ds(col0, SEG)], ssems[b]
        )
        sc.start()
        pending[b] = sc

    b = FULL_PER_W % 2
    col0 = pl.multiple_of(wid * SEG, SEG)
    if pending[b] is not None:
        pending[b].wait()
    gather_rows(NBLK * L, col0, bufs[b])
    tail = []
    for i in range(NTAIL):
        cp = pltpu.make_async_copy(
            bufs[b].at[i],
            outT_hbm.at[NBLK * L + i, pl.ds(col0, SEG)],
            ssems[b],
        )
        cp.start()
        tail.append(cp)

    for cp in tail:
        cp.wait()
    other = pending[1 - b]
    if other is not None:
        other.wait()


def kernel(inputs, categorical_idx):
    outT = _sc_rowgather(inputs.T, categorical_idx)
    return outT.T


# speedup vs baseline: 3.7131x; 1.0829x over previous
import functools

import jax
import jax.numpy as jnp
from jax import lax
from jax.experimental import pallas as pl
from jax.experimental.pallas import tpu as pltpu
from jax.experimental.pallas import tpu_sc as plsc

ROWS = 16384
COLS = 200
NSEL = 100

_info = plsc.get_sparse_core_info()
NC, NS, L = _info.num_cores, _info.num_subcores, _info.num_lanes
NW = NC * NS
NBLK = NSEL // L
NTAIL = NSEL - NBLK * L
SPLIT = 32
SEG = ROWS // SPLIT
FULL_PER_W = NBLK * SPLIT // NW

_mesh = plsc.VectorSubcoreMesh(core_axis_name="c", subcore_axis_name="s")


@functools.partial(
    pl.kernel,
    mesh=_mesh,
    out_type=jax.ShapeDtypeStruct((NSEL, ROWS), jnp.float32),
    scratch_types=[
        pltpu.VMEM((NSEL,), jnp.int32),
        pltpu.VMEM((L, SEG), jnp.float32),
        pltpu.VMEM((L, SEG), jnp.float32),
        pltpu.VMEM((L, SEG), jnp.float32),
        pltpu.VMEM((L, SEG), jnp.float32),
        pltpu.SemaphoreType.DMA,
        pltpu.SemaphoreType.DMA,
        pltpu.SemaphoreType.DMA,
        pltpu.SemaphoreType.DMA,
        pltpu.SemaphoreType.DMA,
        pltpu.SemaphoreType.DMA,
        pltpu.SemaphoreType.DMA,
        pltpu.SemaphoreType.DMA,
    ],
    compiler_params=pltpu.CompilerParams(
        needs_layout_passes=False, use_tc_tiling_on_sc=True
    ),
)
def _sc_rowgather(inT_hbm, idx_hbm, outT_hbm, idx_v,
                  b0, b1, b2, b3, g0, g1, g2, g3, s0, s1, s2, s3):
    wid = lax.axis_index("s") * NC + lax.axis_index("c")
    pltpu.sync_copy(idx_hbm, idx_v)
    lanes = lax.iota(jnp.int32, L)
    bufs = (b0, b1, b2, b3)
    gsems = (g0, g1, g2, g3)
    ssems = (s0, s1, s2, s3)
    NT = FULL_PER_W + 1

    def coords(k):
        if k < FULL_PER_W:
            t = wid * FULL_PER_W + k
            j0 = (t % NBLK) * L
            col0 = pl.multiple_of((t // NBLK) * SEG, SEG)
        else:
            j0 = NBLK * L
            col0 = pl.multiple_of(wid * SEG, SEG)
        return j0, col0

    def start_gather(k):
        j0, col0 = coords(k)
        rows = plsc.load_gather(idx_v, [jnp.minimum(j0 + lanes, NSEL - 1)])
        cp = pltpu.make_async_copy(
            inT_hbm.at[rows, pl.ds(col0, SEG)], bufs[k % 4], gsems[k % 4]
        )
        cp.start()
        return cp

    def start_scatter(k):
        j0, col0 = coords(k)
        b = k % 4
        if k < FULL_PER_W:
            cp = pltpu.make_async_copy(
                bufs[b], outT_hbm.at[pl.ds(j0, L), pl.ds(col0, SEG)], ssems[b]
            )
            cp.start()
            return [cp]
        descs = []
        for i in range(NTAIL):
            cp = pltpu.make_async_copy(
                bufs[b].at[i],
                outT_hbm.at[j0 + i, pl.ds(col0, SEG)],
                ssems[b],
            )
            cp.start()
            descs.append(cp)
        return descs

    gd = {0: start_gather(0), 1: start_gather(1)}
    pend = [None, None, None, None]
    for k in range(NT):
        gd[k].wait()
        pend[k % 4] = start_scatter(k)
        nk = k + 2
        if nk < NT:
            nb = nk % 4
            if pend[nb] is not None:
                for cp in pend[nb]:
                    cp.wait()
                pend[nb] = None
            gd[nk] = start_gather(nk)
    for ds_list in pend:
        if ds_list is not None:
            for cp in ds_list:
                cp.wait()


def kernel(inputs, categorical_idx):
    outT = _sc_rowgather(inputs.T, categorical_idx)
    return outT.T
